# Initial kernel scaffold; baseline (speedup 1.0000x reference)
import jax
import jax.numpy as jnp
from jax.experimental import pallas as pl


def kernel(flat0, flat1, cu0, cu1, x_dense, W0, b0, W1, b1):
    raise NotImplementedError("write your pallas kernel here")


# speedup vs baseline: 65.5653x; 65.5653x over previous
import functools

import jax
import jax.numpy as jnp
from jax import lax
from jax.experimental import pallas as pl
from jax.experimental.pallas import tpu as pltpu
from jax.experimental.pallas import tpu_sc as plsc

_B = 16
_TOT = 16384
_C = 64
_D = 32
_NDENSE = 8

_NW = 32
_ROWS_PER_W = _TOT // _NW
_WORDS_PER_W = _ROWS_PER_W * _C


def _sc_partials_kernel(flat0_hbm, flat1_hbm, cs0_hbm, ce0_hbm, cs1_hbm,
                        ce1_hbm, p0_hbm, p1_hbm, rows_v, acc_v, cs_v, ce_v):
    wid = lax.axis_index("c") * 16 + lax.axis_index("s")
    lo = wid * _ROWS_PER_W
    hi = lo + _ROWS_PER_W

    for flat_hbm, cs_hbm, ce_hbm, p_hbm in (
            (flat0_hbm, cs0_hbm, ce0_hbm, p0_hbm),
            (flat1_hbm, cs1_hbm, ce1_hbm, p1_hbm)):
        word_lo = pl.multiple_of(lo * _C, _WORDS_PER_W)
        pltpu.sync_copy(flat_hbm.at[pl.ds(word_lo, _WORDS_PER_W)], rows_v)
        pltpu.sync_copy(cs_hbm, cs_v)
        pltpu.sync_copy(ce_hbm, ce_v)

        starts = cs_v[...]
        ends = ce_v[...]

        seg_lo = lax.max(starts, jnp.full((16,), 1, jnp.int32) * lo)
        seg_hi = lax.min(ends, jnp.full((16,), 1, jnp.int32) * hi)
        n_v = lax.max(seg_hi - seg_lo, jnp.zeros((16,), jnp.int32))
        base_v = (seg_lo - lo) * _C

        zero = jnp.zeros((16,), jnp.float32)
        for s in range(_B):
            n = n_v[s]
            base = base_v[s]

            def body(i, accs, base=base):
                off = base + i * _C
                return tuple(accs[c] + rows_v[pl.ds(off + 16 * c, 16)]
                             for c in range(4))

            accs = lax.fori_loop(0, n, body, (zero, zero, zero, zero))
            for c in range(4):
                acc_v[pl.ds(s * _C + 16 * c, 16)] = accs[c]

        pltpu.sync_copy(acc_v, p_hbm.at[wid])


def _sc_partials(flat0, flat1, cs0, ce0, cs1, ce1):
    mesh = plsc.VectorSubcoreMesh(core_axis_name="c", subcore_axis_name="s")
    f = pl.kernel(
        _sc_partials_kernel,
        mesh=mesh,
        out_type=[
            jax.ShapeDtypeStruct((_NW, _B * _C), jnp.float32),
            jax.ShapeDtypeStruct((_NW, _B * _C), jnp.float32),
        ],
        scratch_types=[
            pltpu.VMEM((_WORDS_PER_W,), jnp.float32),
            pltpu.VMEM((_B * _C,), jnp.float32),
            pltpu.VMEM((16,), jnp.int32),
            pltpu.VMEM((16,), jnp.int32),
        ],
    )
    return f(flat0.reshape(-1), flat1.reshape(-1), cs0, ce0, cs1, ce1)


def _finish_body(cu0_ref, cu1_ref, p0_ref, p1_ref, xd_ref, w0_ref, b0_ref,
                 w1_ref, b1_ref, o_ref):
    outs = []
    for cu_ref, p_ref, w_ref, b_ref in ((cu0_ref, p0_ref, w0_ref, b0_ref),
                                        (cu1_ref, p1_ref, w1_ref, b1_ref)):
        lmax = cu_ref[1] - cu_ref[0]
        for s in range(1, _B):
            lmax = lax.max(lmax, cu_ref[s + 1] - cu_ref[s])
        scale = 1.0 / lmax.astype(jnp.float32)
        pooled = jnp.sum(p_ref[...], axis=0) * scale
        outs.append(
            jnp.dot(pooled, w_ref[...], preferred_element_type=jnp.float32)
            + b_ref[...])
    o_ref[...] = jnp.concatenate([outs[0], outs[1], xd_ref[...]], axis=-1)


def _finish(cu0, cu1, p0, p1, x_dense, W0, b0, W1, b1):
    smem = pl.BlockSpec(memory_space=pltpu.SMEM)
    return pl.pallas_call(
        _finish_body,
        in_specs=[smem, smem] + [pl.BlockSpec(memory_space=pltpu.VMEM)] * 7,
        out_shape=jax.ShapeDtypeStruct((_B, 2 * _D + _NDENSE), jnp.float32),
    )(cu0, cu1, p0, p1, x_dense, W0, b0.reshape(1, _D), W1,
      b1.reshape(1, _D))


def kernel(flat0, flat1, cu0, cu1, x_dense, W0, b0, W1, b1):
    p0, p1 = _sc_partials(flat0, flat1, cu0[:_B], cu0[1:], cu1[:_B], cu1[1:])
    p0 = p0.reshape(_NW, _B, _C)
    p1 = p1.reshape(_NW, _B, _C)
    return _finish(cu0, cu1, p0, p1, x_dense, W0, b0, W1, b1)


# speedup vs baseline: 88.1728x; 1.3448x over previous
import functools

import jax
import jax.numpy as jnp
from jax import lax
from jax.experimental import pallas as pl
from jax.experimental.pallas import tpu as pltpu
from jax.experimental.pallas import tpu_sc as plsc

_B = 16
_TOT = 16384
_C = 64
_D = 32
_NDENSE = 8

_NW = 32
_ROWS_PER_W = _TOT // _NW
_WORDS_PER_W = _ROWS_PER_W * _C


def _sc_partials_kernel(flat0_hbm, flat1_hbm, cs0_hbm, ce0_hbm, cs1_hbm,
                        ce1_hbm, p0_hbm, p1_hbm, rows_v, acc_v, cs_v, ce_v):
    wid = lax.axis_index("c") * 16 + lax.axis_index("s")
    lo = wid * _ROWS_PER_W
    hi = lo + _ROWS_PER_W

    for flat_hbm, cs_hbm, ce_hbm, p_hbm in (
            (flat0_hbm, cs0_hbm, ce0_hbm, p0_hbm),
            (flat1_hbm, cs1_hbm, ce1_hbm, p1_hbm)):
        row_lo = pl.multiple_of(lo, _ROWS_PER_W)
        pltpu.sync_copy(flat_hbm.at[pl.ds(row_lo, _ROWS_PER_W)], rows_v)
        pltpu.sync_copy(cs_hbm, cs_v)
        pltpu.sync_copy(ce_hbm, ce_v)

        starts = cs_v[...]
        ends = ce_v[...]

        seg_lo = lax.max(starts, jnp.full((16,), 1, jnp.int32) * lo)
        seg_hi = lax.min(ends, jnp.full((16,), 1, jnp.int32) * hi)
        n_v = lax.max(seg_hi - seg_lo, jnp.zeros((16,), jnp.int32))
        base_v = seg_lo - lo

        zero = jnp.zeros((16,), jnp.float32)
        for s in range(_B):
            n = n_v[s]
            base = base_v[s]

            def body(i, accs, base=base):
                r = base + i
                return tuple(accs[c] + rows_v[r, pl.ds(16 * c, 16)]
                             for c in range(4))

            accs = lax.fori_loop(0, n, body, (zero, zero, zero, zero))
            for c in range(4):
                acc_v[s, pl.ds(16 * c, 16)] = accs[c]

        pltpu.sync_copy(acc_v, p_hbm.at[wid])


def _sc_partials(flat0, flat1, cs0, ce0, cs1, ce1):
    mesh = plsc.VectorSubcoreMesh(core_axis_name="c", subcore_axis_name="s")
    f = pl.kernel(
        _sc_partials_kernel,
        mesh=mesh,
        out_type=[
            jax.ShapeDtypeStruct((_NW, _B, _C), jnp.float32),
            jax.ShapeDtypeStruct((_NW, _B, _C), jnp.float32),
        ],
        scratch_types=[
            pltpu.VMEM((_ROWS_PER_W, _C), jnp.float32),
            pltpu.VMEM((_B, _C), jnp.float32),
            pltpu.VMEM((16,), jnp.int32),
            pltpu.VMEM((16,), jnp.int32),
        ],
    )
    return f(flat0, flat1, cs0, ce0, cs1, ce1)


def _finish_body(cu0_ref, cu1_ref, p0_ref, p1_ref, xd_ref, w0_ref, b0_ref,
                 w1_ref, b1_ref, o_ref):
    outs = []
    for cu_ref, p_ref, w_ref, b_ref in ((cu0_ref, p0_ref, w0_ref, b0_ref),
                                        (cu1_ref, p1_ref, w1_ref, b1_ref)):
        lmax = cu_ref[1] - cu_ref[0]
        for s in range(1, _B):
            lmax = lax.max(lmax, cu_ref[s + 1] - cu_ref[s])
        scale = 1.0 / lmax.astype(jnp.float32)
        pooled = jnp.sum(p_ref[...], axis=0) * scale
        outs.append(
            jnp.dot(pooled, w_ref[...], preferred_element_type=jnp.float32)
            + b_ref[...])
    o_ref[...] = jnp.concatenate([outs[0], outs[1], xd_ref[...]], axis=-1)


def _finish(cu0, cu1, p0, p1, x_dense, W0, b0, W1, b1):
    smem = pl.BlockSpec(memory_space=pltpu.SMEM)
    return pl.pallas_call(
        _finish_body,
        in_specs=[smem, smem] + [pl.BlockSpec(memory_space=pltpu.VMEM)] * 7,
        out_shape=jax.ShapeDtypeStruct((_B, 2 * _D + _NDENSE), jnp.float32),
    )(cu0, cu1, p0, p1, x_dense, W0, b0.reshape(1, _D), W1,
      b1.reshape(1, _D))


def kernel(flat0, flat1, cu0, cu1, x_dense, W0, b0, W1, b1):
    p0, p1 = _sc_partials(flat0, flat1, cu0[:_B], cu0[1:], cu1[:_B], cu1[1:])
    return _finish(cu0, cu1, p0, p1, x_dense, W0, b0, W1, b1)


# speedup vs baseline: 122.9258x; 1.3941x over previous
import jax
import jax.numpy as jnp
from jax import lax
from jax.experimental import pallas as pl
from jax.experimental.pallas import tpu as pltpu
from jax.experimental.pallas import tpu_sc as plsc

_B = 16
_TOT = 16384
_C = 64
_D = 32
_NDENSE = 8

_NW = 32
_ROWS_PER_W = _TOT // _NW
_CBLK = 8


def _sc_partials_kernel(f0t_hbm, f1t_hbm, cs0_hbm, ce0_hbm, cs1_hbm,
                        ce1_hbm, p0_hbm, p1_hbm, chunk0_v, chunk1_v, acc_v,
                        trans_v, cs_v, ce_v, av_v, bv_v, sem0, sem1):
    wid = lax.axis_index("c") * 16 + lax.axis_index("s")
    lo = wid * _ROWS_PER_W
    tok_lo = pl.multiple_of(lo, _ROWS_PER_W)
    cp0 = pltpu.async_copy(
        f0t_hbm.at[:, pl.ds(tok_lo, _ROWS_PER_W)], chunk0_v, sem0)
    cp1 = pltpu.async_copy(
        f1t_hbm.at[:, pl.ds(tok_lo, _ROWS_PER_W)], chunk1_v, sem1)

    lane = lax.iota(jnp.int32, 16)
    zero = jnp.zeros((16,), jnp.float32)

    for cs_hbm, ce_hbm, p_hbm, chunk_v, cp in (
            (cs0_hbm, ce0_hbm, p0_hbm, chunk0_v, cp0),
            (cs1_hbm, ce1_hbm, p1_hbm, chunk1_v, cp1)):
        pltpu.sync_copy(cs_hbm, cs_v)
        pltpu.sync_copy(ce_hbm, ce_v)
        cp.wait()

        a_v = jnp.clip(cs_v[...] - lo, 0, _ROWS_PER_W)
        b_v = jnp.clip(ce_v[...] - lo, 0, _ROWS_PER_W)
        av_v[...] = a_v
        bv_v[...] = b_v

        for s in range(_B):
            for k in range(4):
                acc_v[s, pl.ds(16 * k, 16)] = zero

        lane16 = lane * 16

        def seg_body(s, carry, chunk_v=chunk_v):
            a = plsc.load_gather(av_v, [jnp.full((16,), 1, jnp.int32) * s])[0]
            b = plsc.load_gather(bv_v, [jnp.full((16,), 1, jnp.int32) * s])[0]

            @pl.when(b > a)
            def _(a=a, b=b, s=s, chunk_v=chunk_v):
                g_lo = lax.shift_right_logical(a, 4)
                g_hi = lax.shift_right_logical(b + 15, 4)
                for grp in range(4):
                    for half in range(2):
                        c0 = grp * 16 + half * _CBLK

                        def gbody(g, accs, c0=c0, chunk_v=chunk_v, a=a,
                                  b=b):
                            pos = g * 16 + lane
                            m = (pos >= a) & (pos < b)
                            t = g * 16
                            return tuple(
                                accs[j]
                                + jnp.where(m,
                                            chunk_v[c0 + j, pl.ds(t, 16)],
                                            0.0)
                                for j in range(_CBLK))

                        accs = lax.fori_loop(g_lo, g_hi, gbody,
                                             (zero,) * _CBLK)
                        for j in range(_CBLK):
                            plsc.store_scatter(
                                trans_v, [lane16 + (half * _CBLK + j)],
                                accs[j])
                    r = trans_v[pl.ds(0, 16)]
                    for l in range(1, 16):
                        r = r + trans_v[pl.ds(16 * l, 16)]
                    acc_v[s, pl.ds(16 * grp, 16)] = r

            return carry

        lax.fori_loop(0, _B, seg_body, 0)
        pltpu.sync_copy(acc_v, p_hbm.at[wid])


def _sc_partials(f0t, f1t, cs0, ce0, cs1, ce1):
    mesh = plsc.VectorSubcoreMesh(core_axis_name="c", subcore_axis_name="s")
    f = pl.kernel(
        _sc_partials_kernel,
        mesh=mesh,
        compiler_params=pltpu.CompilerParams(needs_layout_passes=False),
        out_type=[
            jax.ShapeDtypeStruct((_NW, _B, _C), jnp.float32),
            jax.ShapeDtypeStruct((_NW, _B, _C), jnp.float32),
        ],
        scratch_types=[
            pltpu.VMEM((_C, _ROWS_PER_W), jnp.float32),
            pltpu.VMEM((_C, _ROWS_PER_W), jnp.float32),
            pltpu.VMEM((_B, _C), jnp.float32),
            pltpu.VMEM((256,), jnp.float32),
            pltpu.VMEM((16,), jnp.int32),
            pltpu.VMEM((16,), jnp.int32),
            pltpu.VMEM((16,), jnp.int32),
            pltpu.VMEM((16,), jnp.int32),
            pltpu.SemaphoreType.DMA,
            pltpu.SemaphoreType.DMA,
        ],
    )
    return f(f0t, f1t, cs0, ce0, cs1, ce1)


def _finish_body(cu0_ref, cu1_ref, p0_ref, p1_ref, xd_ref, w0_ref, b0_ref,
                 w1_ref, b1_ref, o_ref):
    outs = []
    for cu_ref, p_ref, w_ref, b_ref in ((cu0_ref, p0_ref, w0_ref, b0_ref),
                                        (cu1_ref, p1_ref, w1_ref, b1_ref)):
        lmax = cu_ref[1] - cu_ref[0]
        for s in range(1, _B):
            lmax = lax.max(lmax, cu_ref[s + 1] - cu_ref[s])
        scale = 1.0 / lmax.astype(jnp.float32)
        pooled = jnp.sum(p_ref[...], axis=0) * scale
        outs.append(
            jnp.dot(pooled, w_ref[...], preferred_element_type=jnp.float32)
            + b_ref[...])
    o_ref[...] = jnp.concatenate([outs[0], outs[1], xd_ref[...]], axis=-1)


def _finish(cu0, cu1, p0, p1, x_dense, W0, b0, W1, b1):
    smem = pl.BlockSpec(memory_space=pltpu.SMEM)
    return pl.pallas_call(
        _finish_body,
        in_specs=[smem, smem] + [pl.BlockSpec(memory_space=pltpu.VMEM)] * 7,
        out_shape=jax.ShapeDtypeStruct((_B, 2 * _D + _NDENSE), jnp.float32),
    )(cu0, cu1, p0, p1, x_dense, W0, b0.reshape(1, _D), W1,
      b1.reshape(1, _D))


def kernel(flat0, flat1, cu0, cu1, x_dense, W0, b0, W1, b1):
    p0, p1 = _sc_partials(flat0.T, flat1.T, cu0[:_B], cu0[1:], cu1[:_B],
                          cu1[1:])
    return _finish(cu0, cu1, p0, p1, x_dense, W0, b0, W1, b1)


# speedup vs baseline: 123.4022x; 1.0039x over previous
import jax
import jax.numpy as jnp
from jax import lax
from jax.experimental import pallas as pl
from jax.experimental.pallas import tpu as pltpu
from jax.experimental.pallas import tpu_sc as plsc

_B = 16
_TOT = 16384
_C = 64
_D = 32
_NDENSE = 8

_NW = 32
_ROWS_PER_W = _TOT // _NW


def _sc_partials_kernel(f0t_hbm, f1t_hbm, cu0_hbm, cu1_hbm, p0_hbm, p1_hbm,
                        chunk0_v, chunk1_v, acc_v, trans_v, cu_v, av_v,
                        bv_v, sem0, sem1):
    wid = lax.axis_index("c") * 16 + lax.axis_index("s")
    lo = wid * _ROWS_PER_W
    tok_lo = pl.multiple_of(lo, _ROWS_PER_W)
    cp0 = pltpu.async_copy(
        f0t_hbm.at[:, pl.ds(tok_lo, _ROWS_PER_W)], chunk0_v, sem0)
    cp1 = pltpu.async_copy(
        f1t_hbm.at[:, pl.ds(tok_lo, _ROWS_PER_W)], chunk1_v, sem1)

    lane = lax.iota(jnp.int32, 16)
    lane16 = lane * 16
    ones = jnp.full((16,), 1, jnp.int32)
    zero = jnp.zeros((16,), jnp.float32)

    for cu_hbm, p_hbm, chunk_v, cp in ((cu0_hbm, p0_hbm, chunk0_v, cp0),
                                       (cu1_hbm, p1_hbm, chunk1_v, cp1)):
        pltpu.sync_copy(cu_hbm, cu_v)
        starts = plsc.load_gather(cu_v, [lane])
        ends = plsc.load_gather(cu_v, [lane + 1])

        av_v[...] = jnp.clip(starts - lo, 0, _ROWS_PER_W)
        bv_v[...] = jnp.clip(ends - lo, 0, _ROWS_PER_W)

        for s in range(_B):
            for k in range(4):
                acc_v[s, pl.ds(16 * k, 16)] = zero

        cp.wait()

        def seg_body(s, carry, chunk_v=chunk_v):
            a = plsc.load_gather(av_v, [ones * s])[0]
            b = plsc.load_gather(bv_v, [ones * s])[0]

            @pl.when(b > a)
            def _(a=a, b=b, s=s, chunk_v=chunk_v):
                g_lo = lax.shift_right_logical(a + 15, 4)
                g_hi = lax.shift_right_logical(b, 4)
                g_left = lax.shift_right_logical(a, 4)
                t_left = g_left * 16
                t_right = g_hi * 16
                t_right_ld = lax.min(t_right, _ROWS_PER_W - 16)
                pos_l = t_left + lane
                pos_r = t_right + lane
                in_seg_l = (pos_l >= a) & (pos_l < b)
                m_l = in_seg_l & ((pos_l < g_lo * 16) | (pos_l >= t_right))
                m_r = (pos_r >= a) & (pos_r < b) & (g_hi != g_left)

                for grp in range(4):
                    c0 = grp * 16

                    def gbody(g, accs, c0=c0, chunk_v=chunk_v):
                        t = g * 16
                        return tuple(
                            accs[j] + chunk_v[c0 + j, pl.ds(t, 16)]
                            for j in range(16))

                    accs = lax.fori_loop(g_lo, g_hi, gbody, (zero,) * 16)
                    accs = tuple(
                        accs[j]
                        + jnp.where(m_l, chunk_v[c0 + j, pl.ds(t_left, 16)],
                                    0.0)
                        + jnp.where(m_r,
                                    chunk_v[c0 + j, pl.ds(t_right_ld, 16)],
                                    0.0)
                        for j in range(16))
                    for j in range(16):
                        plsc.store_scatter(trans_v, [lane16 + j], accs[j])
                    r = trans_v[pl.ds(0, 16)]
                    for l in range(1, 16):
                        r = r + trans_v[pl.ds(16 * l, 16)]
                    acc_v[s, pl.ds(16 * grp, 16)] = r

            return carry

        lax.fori_loop(0, _B, seg_body, 0)
        pltpu.sync_copy(acc_v, p_hbm.at[wid])


def _sc_partials(f0t, f1t, cu0, cu1):
    mesh = plsc.VectorSubcoreMesh(core_axis_name="c", subcore_axis_name="s")
    f = pl.kernel(
        _sc_partials_kernel,
        mesh=mesh,
        compiler_params=pltpu.CompilerParams(needs_layout_passes=False),
        out_type=[
            jax.ShapeDtypeStruct((_NW, _B, _C), jnp.float32),
            jax.ShapeDtypeStruct((_NW, _B, _C), jnp.float32),
        ],
        scratch_types=[
            pltpu.VMEM((_C, _ROWS_PER_W), jnp.float32),
            pltpu.VMEM((_C, _ROWS_PER_W), jnp.float32),
            pltpu.VMEM((_B, _C), jnp.float32),
            pltpu.VMEM((256,), jnp.float32),
            pltpu.VMEM((_B + 1,), jnp.int32),
            pltpu.VMEM((16,), jnp.int32),
            pltpu.VMEM((16,), jnp.int32),
            pltpu.SemaphoreType.DMA,
            pltpu.SemaphoreType.DMA,
        ],
    )
    return f(f0t, f1t, cu0, cu1)


def _finish_body(cu0_ref, cu1_ref, p0_ref, p1_ref, xd_ref, w0_ref, b0_ref,
                 w1_ref, b1_ref, o_ref):
    outs = []
    for cu_ref, p_ref, w_ref, b_ref in ((cu0_ref, p0_ref, w0_ref, b0_ref),
                                        (cu1_ref, p1_ref, w1_ref, b1_ref)):
        lmax = cu_ref[1] - cu_ref[0]
        for s in range(1, _B):
            lmax = lax.max(lmax, cu_ref[s + 1] - cu_ref[s])
        scale = 1.0 / lmax.astype(jnp.float32)
        pooled = jnp.sum(p_ref[...], axis=0) * scale
        outs.append(
            jnp.dot(pooled, w_ref[...], preferred_element_type=jnp.float32)
            + b_ref[...])
    o_ref[...] = jnp.concatenate([outs[0], outs[1], xd_ref[...]], axis=-1)


def _finish(cu0, cu1, p0, p1, x_dense, W0, b0, W1, b1):
    smem = pl.BlockSpec(memory_space=pltpu.SMEM)
    return pl.pallas_call(
        _finish_body,
        in_specs=[smem, smem] + [pl.BlockSpec(memory_space=pltpu.VMEM)] * 7,
        out_shape=jax.ShapeDtypeStruct((_B, 2 * _D + _NDENSE), jnp.float32),
    )(cu0, cu1, p0, p1, x_dense, W0, b0.reshape(1, _D), W1,
      b1.reshape(1, _D))


def kernel(flat0, flat1, cu0, cu1, x_dense, W0, b0, W1, b1):
    p0, p1 = _sc_partials(flat0.T, flat1.T, cu0, cu1)
    return _finish(cu0, cu1, p0, p1, x_dense, W0, b0, W1, b1)
